# Initial kernel scaffold; baseline (speedup 1.0000x reference)
import jax
import jax.numpy as jnp
from jax.experimental import pallas as pl


def kernel(x, edge_index, edge_attr, W_rel0, b_rel0, W_root0, gamma0, beta0, W_rel1, b_rel1, W_root1, gamma1, beta1, W_rel2, b_rel2, W_root2, gamma2, beta2, W_mlp1, b_mlp1, W_mlp2, b_mlp2):
    raise NotImplementedError("write your pallas kernel here")


# speedup vs baseline: 3.0012x; 3.0012x over previous
import functools

import jax
import jax.numpy as jnp
from jax import lax
from jax.experimental import pallas as pl
from jax.experimental.pallas import tpu as pltpu
from jax.experimental.pallas import tpu_sc as plsc

N = 10000
E = 320000
D = 128
H = 128

NC = 2
NS = 16
NW = NC * NS
BLK = 128
NBLK = E // BLK
BASE = NBLK // NW
REM = NBLK % NW
STRIPE = 624
TAIL = N - NS * STRIPE

_SQRT1_2 = 0.7071067811865476


def _gelu(t):
    return 0.5 * t * (1.0 + lax.erf(t * _SQRT1_2))


def _mm2_body(h_ref, wa_ref, wb_ref, ya_ref, yb_ref):
    h = h_ref[...]
    ya_ref[...] = jnp.dot(h, wa_ref[...], preferred_element_type=jnp.float32)
    yb_ref[...] = jnp.dot(h, wb_ref[...], preferred_element_type=jnp.float32)


def _mm2(h, wa, wb):
    return pl.pallas_call(
        _mm2_body,
        out_shape=[jax.ShapeDtypeStruct((N, H), jnp.float32),
                   jax.ShapeDtypeStruct((N, H), jnp.float32)],
    )(h, wa, wb)


def _post_body(p_ref, r_ref, br_ref, g_ref, b_ref, out_ref):
    t = p_ref[:N, :] + p_ref[N:, :] + r_ref[...] + br_ref[...]
    g = _gelu(t)
    mu = jnp.mean(g, axis=0, keepdims=True)
    d0 = g - mu
    var = jnp.mean(d0 * d0, axis=0, keepdims=True)
    out_ref[...] = d0 / jnp.sqrt(var + 1e-5) * g_ref[...] + b_ref[...]


def _post(p, r, br, gamma, beta):
    return pl.pallas_call(
        _post_body,
        out_shape=jax.ShapeDtypeStruct((N, H), jnp.float32),
    )(p, r, br.reshape(1, H), gamma.reshape(1, H), beta.reshape(1, H))


_FING = 20
_FROWS = 2560
_FCOLS = 125
_FINB = _FROWS // _FING


def _fin_body(ef_ref, b1_ref, w2_ref, b2_ref, out_ref):
    t = ef_ref[...] + b1_ref[...]
    g = _gelu(t)
    v = jnp.sum(g * w2_ref[...], axis=2)
    out_ref[...] = 1.0 / (1.0 + jnp.exp(-(v + b2_ref[...])))


def _fin(ef, b1, w2, b2):
    return pl.pallas_call(
        _fin_body,
        grid=(_FING,),
        in_specs=[
            pl.BlockSpec((_FINB, _FCOLS, H), lambda i: (i, 0, 0)),
            pl.BlockSpec((1, 1, H), lambda i: (0, 0, 0)),
            pl.BlockSpec((1, 1, H), lambda i: (0, 0, 0)),
            pl.BlockSpec((1, 1), lambda i: (0, 0)),
        ],
        out_specs=pl.BlockSpec((_FINB, _FCOLS), lambda i: (i, 0)),
        out_shape=jax.ShapeDtypeStruct((_FROWS, _FCOLS), jnp.float32),
    )(ef, b1.reshape(1, 1, H), w2.reshape(1, 1, H), b2.reshape(1, 1))


_MESH = plsc.VectorSubcoreMesh(core_axis_name="c", subcore_axis_name="s")


def _sc_agg_body(y_hbm, srcb, dstb, ewb, zeros_hbm, out_hbm,
                 acc, src_v, dst_v, ew_v, rows_v):
    cid = lax.axis_index("c")
    sid = lax.axis_index("s")
    wid = cid * NS + sid

    pltpu.sync_copy(zeros_hbm.at[pl.ds(sid * STRIPE, STRIPE)],
                    acc.at[pl.ds(sid * STRIPE, STRIPE)])

    @pl.when(sid == 0)
    def _zero_tail():
        pltpu.sync_copy(zeros_hbm.at[pl.ds(NS * STRIPE, TAIL)],
                        acc.at[pl.ds(NS * STRIPE, TAIL)])

    plsc.subcore_barrier()

    start = wid * BASE + jnp.minimum(wid, REM)
    cnt = BASE + (wid < REM).astype(jnp.int32)

    def chunk(i, _):
        blk = start + i
        pltpu.sync_copy(srcb.at[pl.ds(blk, 1)], src_v)
        pltpu.sync_copy(dstb.at[pl.ds(blk, 1)], dst_v)
        pltpu.sync_copy(ewb.at[pl.ds(blk, 1)], ew_v.at[pl.ds(1, 1)])
        pltpu.sync_copy(y_hbm.at[src_v.at[0]], rows_v)
        one16 = jnp.ones((16,), jnp.int32)
        for e in range(BLK):
            s = plsc.load_gather(ew_v, [one16, jnp.full((16,), e, jnp.int32)])
            for c in range(8):
                sl = pl.ds(c * 16, 16)
                rows_v[e, sl] = rows_v[e, sl] * s
        pltpu.sync_copy(rows_v, acc.at[dst_v.at[0]], add=True)
        return _

    lax.fori_loop(0, cnt, chunk, None)
    plsc.subcore_barrier()
    pltpu.sync_copy(acc.at[pl.ds(sid * STRIPE, STRIPE)],
                    out_hbm.at[pl.ds(cid * N + sid * STRIPE, STRIPE)])

    @pl.when(sid == 0)
    def _read_tail():
        pltpu.sync_copy(acc.at[pl.ds(NS * STRIPE, TAIL)],
                        out_hbm.at[pl.ds(cid * N + NS * STRIPE, TAIL)])


@functools.partial(
    pl.kernel,
    out_type=jax.ShapeDtypeStruct((NC * N, H), jnp.float32),
    mesh=_MESH,
    compiler_params=pltpu.CompilerParams(needs_layout_passes=False),
    scratch_types=[
        pltpu.VMEM_SHARED((N, H), jnp.float32),
        pltpu.VMEM((1, BLK), jnp.int32),
        pltpu.VMEM((1, BLK), jnp.int32),
        pltpu.VMEM((2, BLK), jnp.float32),
        pltpu.VMEM((BLK, H), jnp.float32),
    ],
)
def _sc_agg(y_hbm, srcb, dstb, ewb, zeros_hbm, out_hbm,
            acc, src_v, dst_v, ew_v, rows_v):
    _sc_agg_body(y_hbm, srcb, dstb, ewb, zeros_hbm, out_hbm,
                 acc, src_v, dst_v, ew_v, rows_v)


def _sc_edge_body(a_hbm, b_hbm, srcb, dstb, out_hbm,
                  src_v, dst_v, ra, rb):
    cid = lax.axis_index("c")
    sid = lax.axis_index("s")
    wid = cid * NS + sid
    start = wid * BASE + jnp.minimum(wid, REM)
    cnt = BASE + (wid < REM).astype(jnp.int32)

    def chunk(i, _):
        blk = start + i
        pltpu.sync_copy(srcb.at[pl.ds(blk, 1)], src_v)
        pltpu.sync_copy(dstb.at[pl.ds(blk, 1)], dst_v)
        pltpu.sync_copy(a_hbm.at[src_v.at[0]], ra)
        pltpu.sync_copy(b_hbm.at[dst_v.at[0]], rb)

        def addrow(e, _):
            for c in range(8):
                sl = pl.ds(c * 16, 16)
                ra[e, sl] = ra[e, sl] + rb[e, sl]
            return _

        lax.fori_loop(0, BLK, addrow, None)
        pltpu.sync_copy(ra, out_hbm.at[pl.ds(blk * BLK, BLK)])
        return _

    lax.fori_loop(0, cnt, chunk, None)


@functools.partial(
    pl.kernel,
    out_type=jax.ShapeDtypeStruct((E, H), jnp.float32),
    mesh=_MESH,
    compiler_params=pltpu.CompilerParams(needs_layout_passes=False),
    scratch_types=[
        pltpu.VMEM((1, BLK), jnp.int32),
        pltpu.VMEM((1, BLK), jnp.int32),
        pltpu.VMEM((BLK, H), jnp.float32),
        pltpu.VMEM((BLK, H), jnp.float32),
    ],
)
def _sc_edge(a_hbm, b_hbm, srcb, dstb, out_hbm, src_v, dst_v, ra, rb):
    _sc_edge_body(a_hbm, b_hbm, srcb, dstb, out_hbm, src_v, dst_v, ra, rb)


def kernel(x, edge_index, edge_attr,
           W_rel0, b_rel0, W_root0, gamma0, beta0,
           W_rel1, b_rel1, W_root1, gamma1, beta1,
           W_rel2, b_rel2, W_root2, gamma2, beta2,
           W_mlp1, b_mlp1, W_mlp2, b_mlp2):
    srcb = edge_index[0].reshape(NBLK, BLK)
    dstb = edge_index[1].reshape(NBLK, BLK)
    ewb = edge_attr.reshape(NBLK, BLK)
    zeros = jnp.zeros((N, H), jnp.float32)

    h = x
    for (Wr, br, Wt, g, b) in ((W_rel0, b_rel0, W_root0, gamma0, beta0),
                               (W_rel1, b_rel1, W_root1, gamma1, beta1),
                               (W_rel2, b_rel2, W_root2, gamma2, beta2)):
        y, r = _mm2(h, Wr, Wt)
        p = _sc_agg(y, srcb, dstb, ewb, zeros)
        h = _post(p, r, br, g, b)

    a, bm = _mm2(h, W_mlp1[:H], W_mlp1[H:])
    ef = _sc_edge(a, bm, srcb, dstb)
    out = _fin(ef.reshape(_FROWS, _FCOLS, H), b_mlp1, W_mlp2, b_mlp2)
    return out.reshape(E)
